# baseline (speedup 1.0000x reference)
import functools

import jax
import jax.numpy as jnp
import numpy as np
from jax import lax
from jax.experimental import pallas as pl
from jax.experimental.pallas import tpu as pltpu
from jax.experimental.pallas import tpu_sc as plsc

_N_NODES = 1000000
_NO = 3
_H = 1.0 / (_N_NODES - 1)
_NP = _N_NODES + 2 * _NO
_NQ = 8388608
_FACTOR = float(1.0 / np.sqrt(np.pi))
_INV_H = np.float32(_N_NODES - 1)
_NC2 = np.float32(-(np.float64(_N_NODES - 1) ** 2))

_NC, _NS, _L = 2, 16, 16
_NW = _NC * _NS
_QPW = _NQ // _NW
_V = 1024
_G = 128
_NG = _V // _G
_NT = _QPW // (4 * _V)
_NROWS = (_NP + 2) // 8 - 1


def _sc_body(x_hbm, tab_hbm, y_hbm,
             xa, xb, xc, xd, mv0, mv1, rv0, rv1, rows0, rows1, yv0, yv1,
             gsem0, gsem1, xsa, xsb, xsc, xsd, ysem0, ysem1):
    wid = lax.axis_index("s") * _NC + lax.axis_index("c")
    qbase = wid * _QPW

    def fire_x(b, xv, xsem):
        pltpu.async_copy(x_hbm.at[pl.ds(qbase + b * _V, _V)], xv, xsem)

    def drain_x(xv, xsem):
        pltpu.make_async_copy(x_hbm.at[pl.ds(0, _V)], xv, xsem).wait()

    def stage(xv, xsem, mv, rv, rows, gsem):
        drain_x(xv, xsem)

        lane32 = lax.iota(jnp.int32, _L) * 32

        def p1(j, c):
            x16 = xv[pl.ds(j * _L, _L)]
            idx = (x16 * _INV_H).astype(jnp.int32)
            mv[pl.ds(j * _L, _L)] = idx >> 3
            rv[pl.ds(j * _L, _L)] = (idx & 7) + (lane32 + j * (32 * _L))
            return c

        lax.fori_loop(0, _V // _L, p1, 0, unroll=2)
        for g in range(_NG):
            pltpu.async_copy(
                tab_hbm.at[mv.at[pl.ds(g * _G, _G)]],
                rows.at[pl.ds(g * _G, _G)],
                gsem,
            )

    def drain_rows(rows, gsem):
        pltpu.make_async_copy(tab_hbm.at[pl.ds(0, _V)], rows, gsem).wait()

    def drain_y(yv, ysem):
        pltpu.make_async_copy(x_hbm.at[pl.ds(0, _V)], yv, ysem).wait()

    def pass2(b, xv, rv, rows, yv, ysem):
        zero = jnp.zeros((_L,), jnp.int32)

        def p2(j, c):
            x16 = xv[pl.ds(j * _L, _L)]
            f16 = rv[pl.ds(j * _L, _L)]
            xgs, uus = [], []
            for i in range(7):
                xgs.append(plsc.load_gather(rows, [zero, f16 + i]))
                uus.append(plsc.load_gather(rows, [zero, f16 + (16 + i)]))
            ws = []
            for i in range(7):
                d = x16 - xgs[i]
                ws.append(jnp.exp(d * d * _NC2) * uus[i])
            acc = ((ws[0] + ws[1]) + (ws[2] + ws[3])) + ((ws[4] + ws[5]) + ws[6])
            yv[pl.ds(j * _L, _L)] = acc * np.float32(_FACTOR)
            return c

        lax.fori_loop(0, _V // _L, p2, 0, unroll=2)
        pltpu.async_copy(yv, y_hbm.at[pl.ds(qbase + b * _V, _V)], ysem)

    fire_x(0, xa, xsa)
    fire_x(1, xb, xsb)
    fire_x(2, xc, xsc)
    fire_x(3, xd, xsd)
    stage(xa, xsa, mv0, rv0, rows0, gsem0)

    def quad(t, carry):
        b = 4 * t
        not_last = t + 1 < _NT

        stage(xb, xsb, mv1, rv1, rows1, gsem1)
        drain_rows(rows0, gsem0)

        @pl.when(t > 0)
        def _():
            drain_y(yv0, ysem0)

        pass2(b, xa, rv0, rows0, yv0, ysem0)

        @pl.when(not_last)
        def _():
            fire_x(b + 4, xa, xsa)

        stage(xc, xsc, mv0, rv0, rows0, gsem0)
        drain_rows(rows1, gsem1)

        @pl.when(t > 0)
        def _():
            drain_y(yv1, ysem1)

        pass2(b + 1, xb, rv1, rows1, yv1, ysem1)

        @pl.when(not_last)
        def _():
            fire_x(b + 5, xb, xsb)

        stage(xd, xsd, mv1, rv1, rows1, gsem1)
        drain_rows(rows0, gsem0)
        drain_y(yv0, ysem0)
        pass2(b + 2, xc, rv0, rows0, yv0, ysem0)

        @pl.when(not_last)
        def _():
            fire_x(b + 6, xc, xsc)

        @pl.when(not_last)
        def _():
            stage(xa, xsa, mv0, rv0, rows0, gsem0)

        drain_rows(rows1, gsem1)
        drain_y(yv1, ysem1)
        pass2(b + 3, xd, rv1, rows1, yv1, ysem1)

        @pl.when(not_last)
        def _():
            fire_x(b + 7, xd, xsd)

        return carry

    lax.fori_loop(0, _NT, quad, 0)
    drain_y(yv0, ysem0)
    drain_y(yv1, ysem1)


_fdnet_sc = functools.partial(
    pl.kernel,
    out_type=jax.ShapeDtypeStruct((_NQ,), jnp.float32),
    mesh=plsc.VectorSubcoreMesh(
        core_axis_name="c", subcore_axis_name="s",
        num_cores=_NC, num_subcores=_NS,
    ),
    scratch_types=[
        pltpu.VMEM((_V,), jnp.float32),
        pltpu.VMEM((_V,), jnp.float32),
        pltpu.VMEM((_V,), jnp.float32),
        pltpu.VMEM((_V,), jnp.float32),
        pltpu.VMEM((_V,), jnp.int32),
        pltpu.VMEM((_V,), jnp.int32),
        pltpu.VMEM((_V,), jnp.int32),
        pltpu.VMEM((_V,), jnp.int32),
        pltpu.VMEM((_V, 2 * _L), jnp.float32),
        pltpu.VMEM((_V, 2 * _L), jnp.float32),
        pltpu.VMEM((_V,), jnp.float32),
        pltpu.VMEM((_V,), jnp.float32),
        pltpu.SemaphoreType.DMA,
        pltpu.SemaphoreType.DMA,
        pltpu.SemaphoreType.DMA,
        pltpu.SemaphoreType.DMA,
        pltpu.SemaphoreType.DMA,
        pltpu.SemaphoreType.DMA,
        pltpu.SemaphoreType.DMA,
        pltpu.SemaphoreType.DMA,
    ],
    compiler_params=pltpu.CompilerParams(
        needs_layout_passes=False, use_tc_tiling_on_sc=False,
        disable_bounds_checks=True,
    ),
)(_sc_body)


def kernel(x, u):
    xgrid = jnp.linspace(-_H * _NO, 1.0 + _H * _NO, _NP, dtype=jnp.float32)
    gp = jnp.concatenate([xgrid, jnp.zeros((2,), jnp.float32)])
    up = jnp.concatenate([u, jnp.zeros((2,), jnp.float32)])
    gA = gp[: 16 * (_NROWS // 2)].reshape(_NROWS // 2, 16)
    gB = gp[8: 8 + 16 * (_NROWS // 2)].reshape(_NROWS // 2, 16)
    uA = up[: 16 * (_NROWS // 2)].reshape(_NROWS // 2, 16)
    uB = up[8: 8 + 16 * (_NROWS // 2)].reshape(_NROWS // 2, 16)
    tab = jnp.stack([gA, uA, gB, uB], axis=1).reshape(-1).reshape(_NROWS, 32)
    return _fdnet_sc(x, tab)


# speedup vs baseline: 1.3002x; 1.3002x over previous
import functools

import jax
import jax.numpy as jnp
import numpy as np
from jax import lax
from jax.experimental import pallas as pl
from jax.experimental.pallas import tpu as pltpu
from jax.experimental.pallas import tpu_sc as plsc

_N_NODES = 1000000
_NO = 3
_H = 1.0 / (_N_NODES - 1)
_NP = _N_NODES + 2 * _NO
_NQ = 8388608
_FACTOR = float(1.0 / np.sqrt(np.pi))
_INV_H = np.float32(_N_NODES - 1)
_NC2 = np.float32(-(np.float64(_N_NODES - 1) ** 2))

_NC, _NS, _L = 2, 16, 16
_NW = _NC * _NS
_QPW = _NQ // _NW
_V = 1024
_G = 128
_NG = _V // _G
_NT = _QPW // (4 * _V)
_NROWS = (_NP + 2) // 8 - 1


def _sc_body(x_hbm, tab_hbm, y_hbm,
             xa, xb, xc, xd, mv0, mv1, rv0, rv1, rows0, rows1, yv0, yv1,
             gsem0, gsem1, xsa, xsb, xsc, xsd, ysem0, ysem1):
    wid = lax.axis_index("s") * _NC + lax.axis_index("c")
    qbase = wid * _QPW

    def fire_x(b, xv, xsem):
        pltpu.async_copy(x_hbm.at[pl.ds(qbase + b * _V, _V)], xv, xsem)

    def drain_x(xv, xsem):
        pltpu.make_async_copy(x_hbm.at[pl.ds(0, _V)], xv, xsem).wait()

    def stage(xv, xsem, mv, rv, rows, gsem):
        drain_x(xv, xsem)

        lane32 = lax.iota(jnp.int32, _L) * 32

        def p1(j, c):
            x16 = xv[pl.ds(j * _L, _L)]
            idx = (x16 * _INV_H).astype(jnp.int32)
            mv[pl.ds(j * _L, _L)] = idx >> 3
            rv[pl.ds(j * _L, _L)] = (idx & 7) + (lane32 + j * (32 * _L))
            return c

        lax.fori_loop(0, _V // _L, p1, 0, unroll=2)
        for g in range(_NG):
            pltpu.async_copy(
                tab_hbm.at[mv.at[pl.ds(g * _G, _G)]],
                rows.at[pl.ds(g * _G, _G)],
                gsem,
            )

    def drain_rows(rows, gsem):
        pltpu.make_async_copy(tab_hbm.at[pl.ds(0, _V)], rows, gsem).wait()

    def drain_y(yv, ysem):
        pltpu.make_async_copy(x_hbm.at[pl.ds(0, _V)], yv, ysem).wait()

    def pass2(b, xv, rv, rows, yv, ysem):
        zero = jnp.zeros((_L,), jnp.int32)
        n = _V // _L

        def load_iter(j):
            x16 = xv[pl.ds(j * _L, _L)]
            f16 = rv[pl.ds(j * _L, _L)]
            vals = [x16]
            for i in range(7):
                vals.append(plsc.load_gather(rows, [zero, f16 + i]))
                vals.append(plsc.load_gather(rows, [zero, f16 + (16 + i)]))
            return tuple(vals)

        def p2(j, carry):
            nxt = load_iter(jnp.minimum(j + 1, n - 1))
            x16 = carry[0]
            ws = []
            for i in range(7):
                d = x16 - carry[1 + 2 * i]
                ws.append(jnp.exp(d * d * _NC2) * carry[2 + 2 * i])
            acc = ((ws[0] + ws[1]) + (ws[2] + ws[3])) + ((ws[4] + ws[5]) + ws[6])
            yv[pl.ds(j * _L, _L)] = acc * np.float32(_FACTOR)
            return nxt

        lax.fori_loop(0, n, p2, load_iter(0))
        pltpu.async_copy(yv, y_hbm.at[pl.ds(qbase + b * _V, _V)], ysem)

    fire_x(0, xa, xsa)
    fire_x(1, xb, xsb)
    fire_x(2, xc, xsc)
    fire_x(3, xd, xsd)
    stage(xa, xsa, mv0, rv0, rows0, gsem0)

    def quad(t, carry):
        b = 4 * t
        not_last = t + 1 < _NT

        stage(xb, xsb, mv1, rv1, rows1, gsem1)
        drain_rows(rows0, gsem0)

        @pl.when(t > 0)
        def _():
            drain_y(yv0, ysem0)

        pass2(b, xa, rv0, rows0, yv0, ysem0)

        @pl.when(not_last)
        def _():
            fire_x(b + 4, xa, xsa)

        stage(xc, xsc, mv0, rv0, rows0, gsem0)
        drain_rows(rows1, gsem1)

        @pl.when(t > 0)
        def _():
            drain_y(yv1, ysem1)

        pass2(b + 1, xb, rv1, rows1, yv1, ysem1)

        @pl.when(not_last)
        def _():
            fire_x(b + 5, xb, xsb)

        stage(xd, xsd, mv1, rv1, rows1, gsem1)
        drain_rows(rows0, gsem0)
        drain_y(yv0, ysem0)
        pass2(b + 2, xc, rv0, rows0, yv0, ysem0)

        @pl.when(not_last)
        def _():
            fire_x(b + 6, xc, xsc)

        @pl.when(not_last)
        def _():
            stage(xa, xsa, mv0, rv0, rows0, gsem0)

        drain_rows(rows1, gsem1)
        drain_y(yv1, ysem1)
        pass2(b + 3, xd, rv1, rows1, yv1, ysem1)

        @pl.when(not_last)
        def _():
            fire_x(b + 7, xd, xsd)

        return carry

    lax.fori_loop(0, _NT, quad, 0)
    drain_y(yv0, ysem0)
    drain_y(yv1, ysem1)


_fdnet_sc = functools.partial(
    pl.kernel,
    out_type=jax.ShapeDtypeStruct((_NQ,), jnp.float32),
    mesh=plsc.VectorSubcoreMesh(
        core_axis_name="c", subcore_axis_name="s",
        num_cores=_NC, num_subcores=_NS,
    ),
    scratch_types=[
        pltpu.VMEM((_V,), jnp.float32),
        pltpu.VMEM((_V,), jnp.float32),
        pltpu.VMEM((_V,), jnp.float32),
        pltpu.VMEM((_V,), jnp.float32),
        pltpu.VMEM((_V,), jnp.int32),
        pltpu.VMEM((_V,), jnp.int32),
        pltpu.VMEM((_V,), jnp.int32),
        pltpu.VMEM((_V,), jnp.int32),
        pltpu.VMEM((_V, 2 * _L), jnp.float32),
        pltpu.VMEM((_V, 2 * _L), jnp.float32),
        pltpu.VMEM((_V,), jnp.float32),
        pltpu.VMEM((_V,), jnp.float32),
        pltpu.SemaphoreType.DMA,
        pltpu.SemaphoreType.DMA,
        pltpu.SemaphoreType.DMA,
        pltpu.SemaphoreType.DMA,
        pltpu.SemaphoreType.DMA,
        pltpu.SemaphoreType.DMA,
        pltpu.SemaphoreType.DMA,
        pltpu.SemaphoreType.DMA,
    ],
    compiler_params=pltpu.CompilerParams(
        needs_layout_passes=False, use_tc_tiling_on_sc=False,
        disable_bounds_checks=True,
    ),
)(_sc_body)


def kernel(x, u):
    xgrid = jnp.linspace(-_H * _NO, 1.0 + _H * _NO, _NP, dtype=jnp.float32)
    gp = jnp.concatenate([xgrid, jnp.zeros((2,), jnp.float32)])
    up = jnp.concatenate([u, jnp.zeros((2,), jnp.float32)])
    gA = gp[: 16 * (_NROWS // 2)].reshape(_NROWS // 2, 16)
    gB = gp[8: 8 + 16 * (_NROWS // 2)].reshape(_NROWS // 2, 16)
    uA = up[: 16 * (_NROWS // 2)].reshape(_NROWS // 2, 16)
    uB = up[8: 8 + 16 * (_NROWS // 2)].reshape(_NROWS // 2, 16)
    tab = jnp.stack([gA, uA, gB, uB], axis=1).reshape(-1).reshape(_NROWS, 32)
    return _fdnet_sc(x, tab)


# speedup vs baseline: 1.7983x; 1.3831x over previous
import functools

import jax
import jax.numpy as jnp
import numpy as np
from jax import lax
from jax.experimental import pallas as pl
from jax.experimental.pallas import tpu as pltpu
from jax.experimental.pallas import tpu_sc as plsc

_N_NODES = 1000000
_NO = 3
_H = 1.0 / (_N_NODES - 1)
_NP = _N_NODES + 2 * _NO
_NQ = 8388608
_FACTOR = float(1.0 / np.sqrt(np.pi))
_INV_H = np.float32(_N_NODES - 1)
_NC2 = np.float32(-(np.float64(_N_NODES - 1) ** 2))

_NC, _NS, _L = 2, 16, 16
_NW = _NC * _NS
_QPW = _NQ // _NW
_V = 1024
_G = 128
_NG = _V // _G
_NT = _QPW // (4 * _V)
_NROWS = (_NP + 2) // 8 - 1
_RPW = 7813
_TROWS = 16 * _RPW
_BC = 1024
_BW = 8 * _BC + 16
_GPLEN = 8 * _TROWS + 16


def _sc_body(x_hbm, gp_hbm, up_hbm, y_hbm, tab_hbm,
             xa, xb, xc, xd, mv0, mv1, rv0, rv1, rows0, rows1, yv0, yv1,
             gbuf0, gbuf1, ubuf0, ubuf1,
             gsem0, gsem1, xsa, xsb, xsc, xsd, ysem0, ysem1,
             bg0, bg1, bu0, bu1, bw0, bw1):
    sid = lax.axis_index("s")
    wid = sid * _NC + lax.axis_index("c")
    qbase = wid * _QPW

    wrow = sid * _RPW
    starts = [_BC * c for c in range(_RPW // _BC)] + [_RPW - _BC]
    gbufs, ubufs, gsems, usems = (gbuf0, gbuf1), (ubuf0, ubuf1), (bg0, bg1), (bu0, bu1)
    dsts, wsems = (rows0, rows1), (bw0, bw1)

    def fire_build_in(c):
        s = c & 1
        off = 8 * (wrow + starts[c])
        return (
            pltpu.async_copy(gp_hbm.at[pl.ds(off, _BW)], gbufs[s], gsems[s]),
            pltpu.async_copy(up_hbm.at[pl.ds(off, _BW)], ubufs[s], usems[s]),
        )

    in_descs = {0: fire_build_in(0)}
    w_descs = {}
    for c in range(len(starts)):
        s = c & 1
        if c + 1 < len(starts):
            in_descs[c + 1] = fire_build_in(c + 1)
        for d in in_descs.pop(c):
            d.wait()
        if c >= 2:
            w_descs.pop(c - 2).wait()
        gbuf, ubuf, dst = gbufs[s], ubufs[s], dsts[s]

        def bld(j, carry, gbuf=gbuf, ubuf=ubuf, dst=dst):
            dst[j, pl.ds(0, _L)] = gbuf[pl.ds(8 * j, _L)]
            dst[j, pl.ds(_L, _L)] = ubuf[pl.ds(8 * j, _L)]
            return carry

        lax.fori_loop(0, _BC, bld, 0, unroll=2)
        w_descs[c] = pltpu.async_copy(
            dst, tab_hbm.at[pl.ds(wrow + starts[c], _BC)], wsems[s]
        )
    for d in w_descs.values():
        d.wait()
    plsc.subcore_barrier()

    def fire_x(b, xv, xsem):
        pltpu.async_copy(x_hbm.at[pl.ds(qbase + b * _V, _V)], xv, xsem)

    def drain_x(xv, xsem):
        pltpu.make_async_copy(x_hbm.at[pl.ds(0, _V)], xv, xsem).wait()

    def stage(xv, xsem, mv, rv, rows, gsem):
        drain_x(xv, xsem)

        lane32 = lax.iota(jnp.int32, _L) * 32

        def p1(j, c):
            x16 = xv[pl.ds(j * _L, _L)]
            idx = (x16 * _INV_H).astype(jnp.int32)
            mv[pl.ds(j * _L, _L)] = idx >> 3
            rv[pl.ds(j * _L, _L)] = (idx & 7) + (lane32 + j * (32 * _L))
            return c

        lax.fori_loop(0, _V // _L, p1, 0, unroll=2)
        for g in range(_NG):
            pltpu.async_copy(
                tab_hbm.at[mv.at[pl.ds(g * _G, _G)]],
                rows.at[pl.ds(g * _G, _G)],
                gsem,
            )

    def drain_rows(rows, gsem):
        pltpu.make_async_copy(tab_hbm.at[pl.ds(0, _V)], rows, gsem).wait()

    def drain_y(yv, ysem):
        pltpu.make_async_copy(x_hbm.at[pl.ds(0, _V)], yv, ysem).wait()

    def pass2(b, xv, rv, rows, yv, ysem):
        zero = jnp.zeros((_L,), jnp.int32)
        n = _V // _L

        def load_iter(j):
            x16 = xv[pl.ds(j * _L, _L)]
            f16 = rv[pl.ds(j * _L, _L)]
            vals = [x16]
            for i in range(7):
                vals.append(plsc.load_gather(rows, [zero, f16 + i]))
                vals.append(plsc.load_gather(rows, [zero, f16 + (16 + i)]))
            return tuple(vals)

        def p2(j, carry):
            nxt = load_iter(jnp.minimum(j + 1, n - 1))
            x16 = carry[0]
            ws = []
            for i in range(7):
                d = x16 - carry[1 + 2 * i]
                ws.append(jnp.exp(d * d * _NC2) * carry[2 + 2 * i])
            acc = ((ws[0] + ws[1]) + (ws[2] + ws[3])) + ((ws[4] + ws[5]) + ws[6])
            yv[pl.ds(j * _L, _L)] = acc * np.float32(_FACTOR)
            return nxt

        lax.fori_loop(0, n, p2, load_iter(0))
        pltpu.async_copy(yv, y_hbm.at[pl.ds(qbase + b * _V, _V)], ysem)

    fire_x(0, xa, xsa)
    fire_x(1, xb, xsb)
    fire_x(2, xc, xsc)
    fire_x(3, xd, xsd)
    stage(xa, xsa, mv0, rv0, rows0, gsem0)

    def quad(t, carry):
        b = 4 * t
        not_last = t + 1 < _NT

        stage(xb, xsb, mv1, rv1, rows1, gsem1)
        drain_rows(rows0, gsem0)

        @pl.when(t > 0)
        def _():
            drain_y(yv0, ysem0)

        pass2(b, xa, rv0, rows0, yv0, ysem0)

        @pl.when(not_last)
        def _():
            fire_x(b + 4, xa, xsa)

        stage(xc, xsc, mv0, rv0, rows0, gsem0)
        drain_rows(rows1, gsem1)

        @pl.when(t > 0)
        def _():
            drain_y(yv1, ysem1)

        pass2(b + 1, xb, rv1, rows1, yv1, ysem1)

        @pl.when(not_last)
        def _():
            fire_x(b + 5, xb, xsb)

        stage(xd, xsd, mv1, rv1, rows1, gsem1)
        drain_rows(rows0, gsem0)
        drain_y(yv0, ysem0)
        pass2(b + 2, xc, rv0, rows0, yv0, ysem0)

        @pl.when(not_last)
        def _():
            fire_x(b + 6, xc, xsc)

        @pl.when(not_last)
        def _():
            stage(xa, xsa, mv0, rv0, rows0, gsem0)

        drain_rows(rows1, gsem1)
        drain_y(yv1, ysem1)
        pass2(b + 3, xd, rv1, rows1, yv1, ysem1)

        @pl.when(not_last)
        def _():
            fire_x(b + 7, xd, xsd)

        return carry

    lax.fori_loop(0, _NT, quad, 0)
    drain_y(yv0, ysem0)
    drain_y(yv1, ysem1)


_fdnet_sc = functools.partial(
    pl.kernel,
    out_type=(
        jax.ShapeDtypeStruct((_NQ,), jnp.float32),
        jax.ShapeDtypeStruct((_TROWS, 2 * _L), jnp.float32),
    ),
    mesh=plsc.VectorSubcoreMesh(
        core_axis_name="c", subcore_axis_name="s",
        num_cores=_NC, num_subcores=_NS,
    ),
    scratch_types=[
        pltpu.VMEM((_V,), jnp.float32),
        pltpu.VMEM((_V,), jnp.float32),
        pltpu.VMEM((_V,), jnp.float32),
        pltpu.VMEM((_V,), jnp.float32),
        pltpu.VMEM((_V,), jnp.int32),
        pltpu.VMEM((_V,), jnp.int32),
        pltpu.VMEM((_V,), jnp.int32),
        pltpu.VMEM((_V,), jnp.int32),
        pltpu.VMEM((_V, 2 * _L), jnp.float32),
        pltpu.VMEM((_V, 2 * _L), jnp.float32),
        pltpu.VMEM((_V,), jnp.float32),
        pltpu.VMEM((_V,), jnp.float32),
        pltpu.VMEM((_BW,), jnp.float32),
        pltpu.VMEM((_BW,), jnp.float32),
        pltpu.VMEM((_BW,), jnp.float32),
        pltpu.VMEM((_BW,), jnp.float32),
        pltpu.SemaphoreType.DMA,
        pltpu.SemaphoreType.DMA,
        pltpu.SemaphoreType.DMA,
        pltpu.SemaphoreType.DMA,
        pltpu.SemaphoreType.DMA,
        pltpu.SemaphoreType.DMA,
        pltpu.SemaphoreType.DMA,
        pltpu.SemaphoreType.DMA,
        pltpu.SemaphoreType.DMA,
        pltpu.SemaphoreType.DMA,
        pltpu.SemaphoreType.DMA,
        pltpu.SemaphoreType.DMA,
        pltpu.SemaphoreType.DMA,
        pltpu.SemaphoreType.DMA,
    ],
    compiler_params=pltpu.CompilerParams(
        needs_layout_passes=False, use_tc_tiling_on_sc=False,
        disable_bounds_checks=True,
    ),
)(_sc_body)


def kernel(x, u):
    xgrid = jnp.linspace(-_H * _NO, 1.0 + _H * _NO, _NP, dtype=jnp.float32)
    pad = jnp.zeros((_GPLEN - _NP,), jnp.float32)
    gp = jnp.concatenate([xgrid, pad])
    up = jnp.concatenate([u, pad])
    y, _ = _fdnet_sc(x, gp, up)
    return y


# speedup vs baseline: 1.7983x; 1.0000x over previous
import functools

import jax
import jax.numpy as jnp
import numpy as np
from jax import lax
from jax.experimental import pallas as pl
from jax.experimental.pallas import tpu as pltpu
from jax.experimental.pallas import tpu_sc as plsc

_N_NODES = 1000000
_NO = 3
_H = 1.0 / (_N_NODES - 1)
_NP = _N_NODES + 2 * _NO
_NQ = 8388608
_FACTOR = float(1.0 / np.sqrt(np.pi))
_INV_H = np.float32(_N_NODES - 1)
_NC2 = np.float32(-(np.float64(_N_NODES - 1) ** 2))

_NC, _NS, _L = 2, 16, 16
_NW = _NC * _NS
_QPW = _NQ // _NW
_V = 1024
_G = 128
_NG = _V // _G
_NT = _QPW // (4 * _V)
_NROWS = (_NP + 2) // 8 - 1
_RPW = 7813
_TROWS = 16 * _RPW
_BC = 1024
_BW = 8 * _BC + 16
_GPLEN = 8 * _TROWS + 16


def _sc_body(x_hbm, gp_hbm, up_hbm, y_hbm, tab_hbm,
             xa, xb, xc, xd, mv0, mv1, rv0, rv1, rows0, rows1, yv0, yv1,
             gbuf0, gbuf1, ubuf0, ubuf1,
             gsem0, gsem1, xsa, xsb, xsc, xsd, ysem0, ysem1,
             bg0, bg1, bu0, bu1, bw0, bw1):
    sid = lax.axis_index("s")
    wid = sid * _NC + lax.axis_index("c")
    qbase = wid * _QPW

    wrow = sid * _RPW
    starts = [_BC * c for c in range(_RPW // _BC)] + [_RPW - _BC]
    gbufs, ubufs, gsems, usems = (gbuf0, gbuf1), (ubuf0, ubuf1), (bg0, bg1), (bu0, bu1)
    dsts, wsems = (rows0, rows1), (bw0, bw1)

    def fire_build_in(c):
        s = c & 1
        off = 8 * (wrow + starts[c])
        return (
            pltpu.async_copy(gp_hbm.at[pl.ds(off, _BW)], gbufs[s], gsems[s]),
            pltpu.async_copy(up_hbm.at[pl.ds(off, _BW)], ubufs[s], usems[s]),
        )

    in_descs = {0: fire_build_in(0)}
    w_descs = {}
    for c in range(len(starts)):
        s = c & 1
        if c + 1 < len(starts):
            in_descs[c + 1] = fire_build_in(c + 1)
        for d in in_descs.pop(c):
            d.wait()
        if c >= 2:
            w_descs.pop(c - 2).wait()
        gbuf, ubuf, dst = gbufs[s], ubufs[s], dsts[s]

        def bld(j, carry, gbuf=gbuf, ubuf=ubuf, dst=dst):
            dst[j, pl.ds(0, _L)] = gbuf[pl.ds(8 * j, _L)]
            dst[j, pl.ds(_L, _L)] = ubuf[pl.ds(8 * j, _L)]
            return carry

        lax.fori_loop(0, _BC, bld, 0, unroll=2)
        w_descs[c] = pltpu.async_copy(
            dst, tab_hbm.at[pl.ds(wrow + starts[c], _BC)], wsems[s]
        )
    for d in w_descs.values():
        d.wait()
    plsc.subcore_barrier()

    def fire_x(b, xv, xsem):
        pltpu.async_copy(x_hbm.at[pl.ds(qbase + b * _V, _V)], xv, xsem)

    def drain_x(xv, xsem):
        pltpu.make_async_copy(x_hbm.at[pl.ds(0, _V)], xv, xsem).wait()

    def stage(xv, xsem, mv, rv, rows, gsem):
        drain_x(xv, xsem)

        lane32 = lax.iota(jnp.int32, _L) * 32

        def p1(j, c):
            x16 = xv[pl.ds(j * _L, _L)]
            idx = (x16 * _INV_H).astype(jnp.int32)
            mv[pl.ds(j * _L, _L)] = idx >> 3
            rv[pl.ds(j * _L, _L)] = (idx & 7) + (lane32 + j * (32 * _L))
            return c

        lax.fori_loop(0, _V // _L, p1, 0, unroll=4)
        for g in range(_NG):
            pltpu.async_copy(
                tab_hbm.at[mv.at[pl.ds(g * _G, _G)]],
                rows.at[pl.ds(g * _G, _G)],
                gsem,
            )

    def drain_rows(rows, gsem):
        pltpu.make_async_copy(tab_hbm.at[pl.ds(0, _V)], rows, gsem).wait()

    def drain_y(yv, ysem):
        pltpu.make_async_copy(x_hbm.at[pl.ds(0, _V)], yv, ysem).wait()

    def pass2(b, xv, rv, rows, yv, ysem):
        zero = jnp.zeros((_L,), jnp.int32)
        n = _V // _L

        def load_iter(j):
            x16 = xv[pl.ds(j * _L, _L)]
            f16 = rv[pl.ds(j * _L, _L)]
            vals = [x16]
            for i in range(7):
                vals.append(plsc.load_gather(rows, [zero, f16 + i]))
                vals.append(plsc.load_gather(rows, [zero, f16 + (16 + i)]))
            return tuple(vals)

        def p2(j, carry):
            nxt = load_iter(jnp.minimum(j + 1, n - 1))
            x16 = carry[0]
            ws = []
            for i in range(7):
                d = x16 - carry[1 + 2 * i]
                ws.append(jnp.exp(d * d * _NC2) * carry[2 + 2 * i])
            acc = ((ws[0] + ws[1]) + (ws[2] + ws[3])) + ((ws[4] + ws[5]) + ws[6])
            yv[pl.ds(j * _L, _L)] = acc * np.float32(_FACTOR)
            return nxt

        lax.fori_loop(0, n, p2, load_iter(0))
        pltpu.async_copy(yv, y_hbm.at[pl.ds(qbase + b * _V, _V)], ysem)

    fire_x(0, xa, xsa)
    fire_x(1, xb, xsb)
    fire_x(2, xc, xsc)
    fire_x(3, xd, xsd)
    stage(xa, xsa, mv0, rv0, rows0, gsem0)

    def quad(t, carry):
        b = 4 * t
        not_last = t + 1 < _NT

        stage(xb, xsb, mv1, rv1, rows1, gsem1)
        drain_rows(rows0, gsem0)

        @pl.when(t > 0)
        def _():
            drain_y(yv0, ysem0)

        pass2(b, xa, rv0, rows0, yv0, ysem0)

        @pl.when(not_last)
        def _():
            fire_x(b + 4, xa, xsa)

        stage(xc, xsc, mv0, rv0, rows0, gsem0)
        drain_rows(rows1, gsem1)

        @pl.when(t > 0)
        def _():
            drain_y(yv1, ysem1)

        pass2(b + 1, xb, rv1, rows1, yv1, ysem1)

        @pl.when(not_last)
        def _():
            fire_x(b + 5, xb, xsb)

        stage(xd, xsd, mv1, rv1, rows1, gsem1)
        drain_rows(rows0, gsem0)
        drain_y(yv0, ysem0)
        pass2(b + 2, xc, rv0, rows0, yv0, ysem0)

        @pl.when(not_last)
        def _():
            fire_x(b + 6, xc, xsc)

        @pl.when(not_last)
        def _():
            stage(xa, xsa, mv0, rv0, rows0, gsem0)

        drain_rows(rows1, gsem1)
        drain_y(yv1, ysem1)
        pass2(b + 3, xd, rv1, rows1, yv1, ysem1)

        @pl.when(not_last)
        def _():
            fire_x(b + 7, xd, xsd)

        return carry

    lax.fori_loop(0, _NT, quad, 0)
    drain_y(yv0, ysem0)
    drain_y(yv1, ysem1)


_fdnet_sc = functools.partial(
    pl.kernel,
    out_type=(
        jax.ShapeDtypeStruct((_NQ,), jnp.float32),
        jax.ShapeDtypeStruct((_TROWS, 2 * _L), jnp.float32),
    ),
    mesh=plsc.VectorSubcoreMesh(
        core_axis_name="c", subcore_axis_name="s",
        num_cores=_NC, num_subcores=_NS,
    ),
    scratch_types=[
        pltpu.VMEM((_V,), jnp.float32),
        pltpu.VMEM((_V,), jnp.float32),
        pltpu.VMEM((_V,), jnp.float32),
        pltpu.VMEM((_V,), jnp.float32),
        pltpu.VMEM((_V,), jnp.int32),
        pltpu.VMEM((_V,), jnp.int32),
        pltpu.VMEM((_V,), jnp.int32),
        pltpu.VMEM((_V,), jnp.int32),
        pltpu.VMEM((_V, 2 * _L), jnp.float32),
        pltpu.VMEM((_V, 2 * _L), jnp.float32),
        pltpu.VMEM((_V,), jnp.float32),
        pltpu.VMEM((_V,), jnp.float32),
        pltpu.VMEM((_BW,), jnp.float32),
        pltpu.VMEM((_BW,), jnp.float32),
        pltpu.VMEM((_BW,), jnp.float32),
        pltpu.VMEM((_BW,), jnp.float32),
        pltpu.SemaphoreType.DMA,
        pltpu.SemaphoreType.DMA,
        pltpu.SemaphoreType.DMA,
        pltpu.SemaphoreType.DMA,
        pltpu.SemaphoreType.DMA,
        pltpu.SemaphoreType.DMA,
        pltpu.SemaphoreType.DMA,
        pltpu.SemaphoreType.DMA,
        pltpu.SemaphoreType.DMA,
        pltpu.SemaphoreType.DMA,
        pltpu.SemaphoreType.DMA,
        pltpu.SemaphoreType.DMA,
        pltpu.SemaphoreType.DMA,
        pltpu.SemaphoreType.DMA,
    ],
    compiler_params=pltpu.CompilerParams(
        needs_layout_passes=False, use_tc_tiling_on_sc=False,
        disable_bounds_checks=True,
    ),
)(_sc_body)


def kernel(x, u):
    xgrid = jnp.linspace(-_H * _NO, 1.0 + _H * _NO, _NP, dtype=jnp.float32)
    pad = jnp.zeros((_GPLEN - _NP,), jnp.float32)
    gp = jnp.concatenate([xgrid, pad])
    up = jnp.concatenate([u, pad])
    y, _ = _fdnet_sc(x, gp, up)
    return y
